# baseline (speedup 1.0000x reference)
import functools

import jax
import jax.numpy as jnp
from jax import lax
from jax.experimental import pallas as pl
from jax.experimental.pallas import tpu as pltpu
from jax.experimental.pallas import tpu_sc as plsc

_NUM_CORES = 2
_NUM_SUBCORES = 16
_NUM_WORKERS = _NUM_CORES * _NUM_SUBCORES

_CHUNK = 40


@functools.partial(jax.jit, static_argnames=("n_rows", "d"))
def _gather_rows(table, idx_flat, n_rows, d):
    n_per_w = n_rows // _NUM_WORKERS
    n_chunks = n_per_w // _CHUNK
    n_pairs = n_chunks // 2
    mesh = plsc.VectorSubcoreMesh(core_axis_name="c", subcore_axis_name="s")

    @functools.partial(
        pl.kernel,
        mesh=mesh,
        compiler_params=pltpu.CompilerParams(use_tc_tiling_on_sc=False),
        out_type=jax.ShapeDtypeStruct((n_rows, d), jnp.float32),
        scratch_types=[
            pltpu.VMEM((n_per_w,), jnp.int32),
            pltpu.VMEM((_CHUNK, d), jnp.float32),
            pltpu.VMEM((_CHUNK, d), jnp.float32),
            pltpu.SemaphoreType.DMA,
            pltpu.SemaphoreType.DMA,
            pltpu.SemaphoreType.DMA,
            pltpu.SemaphoreType.DMA,
        ],
    )
    def k(table_hbm, idx_hbm, out_hbm, idx_v, b0, b1, gs0, gs1, ss0, ss1):
        wid = lax.axis_index("s") * _NUM_CORES + lax.axis_index("c")
        base = wid * n_per_w
        pltpu.sync_copy(idx_hbm.at[pl.ds(base, n_per_w)], idx_v)

        def gather(c, buf, sem):
            pltpu.async_copy(
                table_hbm.at[idx_v.at[pl.ds(c * _CHUNK, _CHUNK)]], buf, sem
            )

        def scatter(buf, c, sem):
            pltpu.async_copy(
                buf, out_hbm.at[pl.ds(base + c * _CHUNK, _CHUNK)], sem
            )

        def wait_gather(buf, sem):
            pltpu.make_async_copy(table_hbm.at[pl.ds(0, _CHUNK)], buf, sem).wait()

        def wait_scatter(buf, sem):
            pltpu.make_async_copy(
                buf, out_hbm.at[pl.ds(base, _CHUNK)], sem
            ).wait()

        gather(0, b0, gs0)

        def body(p, carry):
            a = 2 * p
            wait_gather(b0, gs0)
            scatter(b0, a, ss0)

            @pl.when(p > 0)
            def _():
                wait_scatter(b1, ss1)

            gather(a + 1, b1, gs1)
            wait_gather(b1, gs1)
            scatter(b1, a + 1, ss1)
            wait_scatter(b0, ss0)

            @pl.when(p < n_pairs - 1)
            def _():
                gather(a + 2, b0, gs0)

            return carry

        lax.fori_loop(0, n_pairs, body, 0)
        wait_scatter(b1, ss1)

    return k(table, idx_flat)


def kernel(table, idx):
    v, d = table.shape
    b, t = idx.shape
    out = _gather_rows(table, idx.reshape(b * t), b * t, d)
    return out.reshape(b, t, v)


# speedup vs baseline: 1.0012x; 1.0012x over previous
import functools

import jax
import jax.numpy as jnp
from jax import lax
from jax.experimental import pallas as pl
from jax.experimental.pallas import tpu as pltpu
from jax.experimental.pallas import tpu_sc as plsc

_NUM_CORES = 2
_NUM_SUBCORES = 16
_NUM_WORKERS = _NUM_CORES * _NUM_SUBCORES

_CHUNK = 16
_NBUF = 5


@functools.partial(jax.jit, static_argnames=("n_rows", "d"))
def _gather_rows(table, idx_flat, n_rows, d):
    n_per_w = n_rows // _NUM_WORKERS
    n_chunks = n_per_w // _CHUNK
    n_groups = n_chunks // _NBUF
    mesh = plsc.VectorSubcoreMesh(core_axis_name="c", subcore_axis_name="s")

    @functools.partial(
        pl.kernel,
        mesh=mesh,
        compiler_params=pltpu.CompilerParams(use_tc_tiling_on_sc=False),
        out_type=jax.ShapeDtypeStruct((n_rows, d), jnp.float32),
        scratch_types=[
            pltpu.VMEM((n_per_w,), jnp.int32),
            [pltpu.VMEM((_CHUNK, d), jnp.float32) for _ in range(_NBUF)],
            [pltpu.SemaphoreType.DMA for _ in range(_NBUF)],
            [pltpu.SemaphoreType.DMA for _ in range(_NBUF)],
        ],
    )
    def k(table_hbm, idx_hbm, out_hbm, idx_v, bufs, gsems, ssems):
        wid = lax.axis_index("s") * _NUM_CORES + lax.axis_index("c")
        base = wid * n_per_w
        pltpu.sync_copy(idx_hbm.at[pl.ds(base, n_per_w)], idx_v)

        def gather(c, buf, sem):
            pltpu.async_copy(
                table_hbm.at[idx_v.at[pl.ds(c * _CHUNK, _CHUNK)]], buf, sem
            )

        def scatter(buf, c, sem):
            pltpu.async_copy(
                buf, out_hbm.at[pl.ds(base + c * _CHUNK, _CHUNK)], sem
            )

        def wait_gather(buf, sem):
            pltpu.make_async_copy(table_hbm.at[pl.ds(0, _CHUNK)], buf, sem).wait()

        def wait_scatter(buf, sem):
            pltpu.make_async_copy(
                buf, out_hbm.at[pl.ds(base, _CHUNK)], sem
            ).wait()

        for b in range(_NBUF):
            gather(b, bufs[b], gsems[b])

        def body(g, carry):
            c0 = g * _NBUF
            for b in range(_NBUF):
                wait_gather(bufs[b], gsems[b])
                scatter(bufs[b], c0 + b, ssems[b])
            for b in range(_NBUF):
                wait_scatter(bufs[b], ssems[b])

                @pl.when(g < n_groups - 1)
                def _(b=b):
                    gather(c0 + b + _NBUF, bufs[b], gsems[b])

            return carry

        lax.fori_loop(0, n_groups, body, 0)

    return k(table, idx_flat)


def kernel(table, idx):
    v, d = table.shape
    b, t = idx.shape
    out = _gather_rows(table, idx.reshape(b * t), b * t, d)
    return out.reshape(b, t, v)


# speedup vs baseline: 1.1111x; 1.1099x over previous
import functools

import jax
import jax.numpy as jnp
from jax import lax
from jax.experimental import pallas as pl
from jax.experimental.pallas import tpu as pltpu
from jax.experimental.pallas import tpu_sc as plsc

_NUM_CORES = 2
_NUM_SUBCORES = 16

_W = 512
_CHUNK = 16
_NBUF = 5
_ROWS_PER_TILE = 63


@functools.partial(jax.jit, static_argnames=("n_rows", "d"))
def _gather_rows(table, idx_flat, n_rows, d):
    v = table.shape[0]
    n_per_t = n_rows // _NUM_SUBCORES
    n_chunks = n_per_t // _CHUNK
    n_groups = n_chunks // _NBUF
    mesh = plsc.VectorSubcoreMesh(core_axis_name="c", subcore_axis_name="s")

    @functools.partial(
        pl.kernel,
        mesh=mesh,
        compiler_params=pltpu.CompilerParams(use_tc_tiling_on_sc=False),
        out_type=jax.ShapeDtypeStruct((n_rows, d), jnp.float32),
        scratch_types=[
            pltpu.VMEM((n_per_t,), jnp.int32),
            pltpu.VMEM_SHARED((16 * _ROWS_PER_TILE, _W), jnp.float32),
            [pltpu.VMEM((_CHUNK, _W), jnp.float32) for _ in range(_NBUF)],
            [pltpu.SemaphoreType.DMA for _ in range(_NBUF)],
            [pltpu.SemaphoreType.DMA for _ in range(_NBUF)],
        ],
    )
    def k(table_hbm, idx_hbm, out_hbm, idx_v, shared, bufs, gsems, ssems):
        c = lax.axis_index("c")
        s = lax.axis_index("s")
        col0 = c * (d - _W)
        pos0 = s * n_per_t
        pltpu.sync_copy(idx_hbm.at[pl.ds(pos0, n_per_t)], idx_v)

        r0 = s * _ROWS_PER_TILE
        full = jnp.minimum(r0 + _ROWS_PER_TILE, v) - r0 == _ROWS_PER_TILE

        @pl.when(full)
        def _():
            pltpu.sync_copy(
                table_hbm.at[pl.ds(r0, _ROWS_PER_TILE), pl.ds(col0, _W)],
                shared.at[pl.ds(r0, _ROWS_PER_TILE)],
            )

        rem = v - (v // _ROWS_PER_TILE) * _ROWS_PER_TILE

        @pl.when(jnp.logical_not(full) & (r0 < v))
        def _():
            pltpu.sync_copy(
                table_hbm.at[pl.ds(v - rem, rem), pl.ds(col0, _W)],
                shared.at[pl.ds(v - rem, rem)],
            )

        plsc.subcore_barrier()

        def gather(ch, buf, sem):
            pltpu.async_copy(
                shared.at[idx_v.at[pl.ds(ch * _CHUNK, _CHUNK)]], buf, sem
            )

        def scatter(buf, ch, sem):
            pltpu.async_copy(
                buf,
                out_hbm.at[pl.ds(pos0 + ch * _CHUNK, _CHUNK), pl.ds(col0, _W)],
                sem,
            )

        def wait_gather(buf, sem):
            pltpu.make_async_copy(shared.at[pl.ds(0, _CHUNK)], buf, sem).wait()

        def wait_scatter(buf, sem):
            pltpu.make_async_copy(
                buf, out_hbm.at[pl.ds(pos0, _CHUNK), pl.ds(col0, _W)], sem
            ).wait()

        for b in range(_NBUF):
            gather(b, bufs[b], gsems[b])

        def body(g, carry):
            c0 = g * _NBUF
            for b in range(_NBUF):
                wait_gather(bufs[b], gsems[b])
                scatter(bufs[b], c0 + b, ssems[b])
            for b in range(_NBUF):
                wait_scatter(bufs[b], ssems[b])

                @pl.when(g < n_groups - 1)
                def _(b=b):
                    gather(c0 + b + _NBUF, bufs[b], gsems[b])

            return carry

        lax.fori_loop(0, n_groups, body, 0)

    return k(table, idx_flat)


def kernel(table, idx):
    v, d = table.shape
    b, t = idx.shape
    out = _gather_rows(table, idx.reshape(b * t), b * t, d)
    return out.reshape(b, t, v)
